# baseline (speedup 1.0000x reference)
import functools

import jax
import jax.numpy as jnp
from jax import lax
from jax.experimental import pallas as pl
from jax.experimental.pallas import tpu as pltpu
from jax.experimental.pallas import tpu_sc as plsc

_NC = 2
_NS = 16
_CHUNK = 128
_BR = 1024
_G = 64


def _sc_mesh():
    return plsc.VectorSubcoreMesh(core_axis_name="c", subcore_axis_name="s",
                                  num_cores=_NC, num_subcores=_NS)


def _make_deg_kernel(NP, NCH):
    rpt = NP // _NS

    @functools.partial(
        pl.kernel,
        out_type=jax.ShapeDtypeStruct((_NC, NP), jnp.float32),
        mesh=_sc_mesh(),
        compiler_params=pltpu.CompilerParams(use_tc_tiling_on_sc=False, skip_device_barrier=True, disable_bounds_checks=True, disable_semaphore_checks=True),
        scratch_types=[
            pltpu.VMEM((NCH, _CHUNK), jnp.int32),
            pltpu.VMEM((_CHUNK,), jnp.float32),
            pltpu.VMEM((rpt,), jnp.float32),
            pltpu.VMEM_SHARED((NP,), jnp.float32),
        ],
    )
    def deg_kernel(d_hbm, out_hbm, didx, ones, zbuf, acc):
        c = lax.axis_index("c")
        t = lax.axis_index("s")
        pltpu.sync_copy(d_hbm.at[c, t], didx)
        one16 = jnp.ones((16,), jnp.float32)
        zero16 = jnp.zeros((16,), jnp.float32)
        for i in range(_CHUNK // 16):
            ones[pl.ds(i * 16, 16)] = one16
        for i in range(rpt // 16):
            zbuf[pl.ds(i * 16, 16)] = zero16
        pltpu.sync_copy(zbuf, acc.at[pl.ds(t * rpt, rpt)])
        plsc.subcore_barrier()

        def body(j, carry):
            pltpu.sync_copy(ones, acc.at[didx.at[j]], add=True)
            return carry

        lax.fori_loop(0, NCH, body, 0, unroll=False)
        plsc.subcore_barrier()
        pltpu.sync_copy(acc.at[pl.ds(t * rpt, rpt)],
                        out_hbm.at[c, pl.ds(t * rpt, rpt)])

    return deg_kernel


def _make_agg_kernel(NP, NCH, H):
    rpt = NP // _NS

    @functools.partial(
        pl.kernel,
        out_type=jax.ShapeDtypeStruct((_NC, NP, H), jnp.float32),
        mesh=_sc_mesh(),
        compiler_params=pltpu.CompilerParams(use_tc_tiling_on_sc=False, skip_device_barrier=True, disable_bounds_checks=True, disable_semaphore_checks=True),
        scratch_types=[
            pltpu.VMEM((NCH, _CHUNK), jnp.int32),
            pltpu.VMEM((NCH, _CHUNK), jnp.int32),
            pltpu.VMEM((8, _CHUNK, H), jnp.float32),
            pltpu.VMEM_SHARED((NP, H), jnp.float32),
            [pltpu.SemaphoreType.DMA] * 8,
            [pltpu.SemaphoreType.DMA] * 8,
            pltpu.SemaphoreType.DMA,
        ],
    )
    def agg_kernel(hs_hbm, s_hbm, d_hbm, out_hbm, sidx, didx, rows, acc,
                   gsem, ssem, isem):
        c = lax.axis_index("c")
        t = lax.axis_index("s")

        def gather(j, b):
            return pltpu.make_async_copy(hs_hbm.at[sidx.at[j]], rows.at[b],
                                         gsem[b])

        def scatter_start(j, b):
            pltpu.async_copy(rows.at[b], acc.at[didx.at[j]], ssem[b],
                             add=True)

        def scatter_wait(j, b):
            pltpu.make_async_copy(rows.at[b], acc.at[didx.at[j]],
                                  ssem[b]).wait()

        pltpu.sync_copy(s_hbm.at[c, t], sidx)
        init_cp = pltpu.make_async_copy(hs_hbm.at[pl.ds(t * rpt, rpt)],
                                        acc.at[pl.ds(t * rpt, rpt)], isem)
        init_cp.start()
        for b in range(4):
            gather(b, b).start()
        pltpu.sync_copy(d_hbm.at[c, t], didx)
        init_cp.wait()
        plsc.subcore_barrier()

        def body(jj, carry):
            for b in range(8):
                j = 8 * jj + b
                gather(j, b).wait()
                scatter_start(j, b)
                b4 = (b + 4) % 8
                j4 = j + 4

                @pl.when(j4 >= 8)
                def _():
                    scatter_wait(j4 - 8, b4)

                @pl.when(j4 < NCH)
                def _():
                    gather(j4, b4).start()
            return carry

        lax.fori_loop(0, NCH // 8, body, 0, unroll=False)
        for k in range(4, 0, -1):
            scatter_wait(NCH - k, (NCH - k) % 8)
        plsc.subcore_barrier()
        pltpu.sync_copy(acc.at[pl.ds(t * rpt, rpt)],
                        out_hbm.at[c, pl.ds(t * rpt, rpt)])

    return agg_kernel


def _mm_body(x_ref, w_ref, h_ref):
    h_ref[...] = jnp.dot(x_ref[...], w_ref[...],
                         preferred_element_type=jnp.float32)


def _scale_body(h_ref, degp_ref, hs_ref, dinv_ref, *, Nreal):
    i = pl.program_id(0)
    deg = degp_ref[:, 0:1] + degp_ref[:, 1:2] + 1.0
    dinv = lax.rsqrt(deg)
    rows = i * _BR + lax.broadcasted_iota(jnp.int32, (_BR, 1), 0)
    valid = rows < Nreal
    hs_ref[...] = jnp.where(valid, dinv * h_ref[...], 0.0)
    dinv_ref[...] = jnp.where(valid, dinv, 1.0)


def _mid_body(p_ref, hs_ref, dinv_ref, b_ref, w_ref, out_ref, *, Nreal):
    i = pl.program_id(0)
    z = dinv_ref[...] * (p_ref[0] + p_ref[1] - hs_ref[...]) + b_ref[...]
    h = jnp.maximum(z, 0.0)
    hn = jnp.dot(h, w_ref[...], preferred_element_type=jnp.float32)
    rows = i * _BR + lax.broadcasted_iota(jnp.int32, (_BR, 1), 0)
    out_ref[...] = jnp.where(rows < Nreal, dinv_ref[...] * hn, 0.0)


def _final_body(p_ref, hs_ref, dinv_ref, b_ref, batch_ref, wfc_ref, bfc_ref,
                out_ref, pooled_acc, cnt_acc, *, nblocks, G):
    i = pl.program_id(0)

    @pl.when(i == 0)
    def _():
        pooled_acc[...] = jnp.zeros_like(pooled_acc)
        cnt_acc[...] = jnp.zeros_like(cnt_acc)

    z = dinv_ref[...] * (p_ref[0] + p_ref[1] - hs_ref[...]) + b_ref[...]
    gids = lax.broadcasted_iota(jnp.int32, (G, _BR), 0)
    mask = (gids == batch_ref[...]).astype(jnp.float32)
    pooled_acc[...] += jnp.dot(mask, z, preferred_element_type=jnp.float32)
    cnt_acc[...] += jnp.sum(mask, axis=1, keepdims=True)

    @pl.when(i == nblocks - 1)
    def _():
        pooled = pooled_acc[...] / jnp.maximum(cnt_acc[...], 1.0)
        out_ref[...] = (jnp.dot(pooled, wfc_ref[...],
                                preferred_element_type=jnp.float32)
                        + bfc_ref[...])


def kernel(x, edge_index, batch, W1, b1, W2, b2, W3, b3, Wfc, bfc):
    N, D = x.shape
    E = edge_index.shape[1]
    H = W1.shape[1]
    C = Wfc.shape[1]
    G = _G

    NP = ((N + 1 + _BR - 1) // _BR) * _BR
    nblocks = NP // _BR
    per_round = _NC * _NS * _CHUNK
    NCH = -(-E // per_round)
    NCH = ((NCH + 7) // 8) * 8
    EP = NCH * per_round
    pad = EP - E
    npad_rows = NP - N

    spread = (jnp.arange(pad, dtype=jnp.int32) % npad_rows) + N
    s_arr = jnp.concatenate([edge_index[0], spread]).reshape(
        _NC, _NS, NCH, _CHUNK)
    d_arr = jnp.concatenate([edge_index[1], spread]).reshape(
        _NC, _NS, NCH, _CHUNK)
    batch_p = jnp.concatenate(
        [batch, jnp.full((NP - N,), G, jnp.int32)]).reshape(1, NP)

    deg_kernel = _make_deg_kernel(NP, NCH)
    agg_kernel = _make_agg_kernel(NP, NCH, H)

    degp = deg_kernel(d_arr)
    degp_t = degp.T

    h1 = pl.pallas_call(
        _mm_body,
        grid=(nblocks,),
        in_specs=[
            pl.BlockSpec((_BR, D), lambda i: (i, 0)),
            pl.BlockSpec((D, H), lambda i: (0, 0)),
        ],
        out_specs=pl.BlockSpec((_BR, H), lambda i: (i, 0)),
        out_shape=jax.ShapeDtypeStruct((NP, H), jnp.float32),
    )(x, W1)

    hs1, dinv = pl.pallas_call(
        functools.partial(_scale_body, Nreal=N),
        grid=(nblocks,),
        in_specs=[
            pl.BlockSpec((_BR, H), lambda i: (i, 0)),
            pl.BlockSpec((_BR, 2), lambda i: (i, 0)),
        ],
        out_specs=[
            pl.BlockSpec((_BR, H), lambda i: (i, 0)),
            pl.BlockSpec((_BR, 1), lambda i: (i, 0)),
        ],
        out_shape=[
            jax.ShapeDtypeStruct((NP, H), jnp.float32),
            jax.ShapeDtypeStruct((NP, 1), jnp.float32),
        ],
    )(h1, degp_t)

    def mid(p, hs, b, W):
        return pl.pallas_call(
            functools.partial(_mid_body, Nreal=N),
            grid=(nblocks,),
            in_specs=[
                pl.BlockSpec((_NC, _BR, H), lambda i: (0, i, 0)),
                pl.BlockSpec((_BR, H), lambda i: (i, 0)),
                pl.BlockSpec((_BR, 1), lambda i: (i, 0)),
                pl.BlockSpec((1, H), lambda i: (0, 0)),
                pl.BlockSpec((H, H), lambda i: (0, 0)),
            ],
            out_specs=pl.BlockSpec((_BR, H), lambda i: (i, 0)),
            out_shape=jax.ShapeDtypeStruct((NP, H), jnp.float32),
        )(p, hs, dinv, b.reshape(1, H), W)

    p1 = agg_kernel(hs1, s_arr, d_arr)
    hs2 = mid(p1, hs1, b1, W2)
    p2 = agg_kernel(hs2, s_arr, d_arr)
    hs3 = mid(p2, hs2, b2, W3)
    p3 = agg_kernel(hs3, s_arr, d_arr)

    out = pl.pallas_call(
        functools.partial(_final_body, nblocks=nblocks, G=G),
        grid=(nblocks,),
        in_specs=[
            pl.BlockSpec((_NC, _BR, H), lambda i: (0, i, 0)),
            pl.BlockSpec((_BR, H), lambda i: (i, 0)),
            pl.BlockSpec((_BR, 1), lambda i: (i, 0)),
            pl.BlockSpec((1, H), lambda i: (0, 0)),
            pl.BlockSpec((1, _BR), lambda i: (0, i)),
            pl.BlockSpec((H, C), lambda i: (0, 0)),
            pl.BlockSpec((1, C), lambda i: (0, 0)),
        ],
        out_specs=pl.BlockSpec((G, C), lambda i: (0, 0)),
        out_shape=jax.ShapeDtypeStruct((G, C), jnp.float32),
        scratch_shapes=[
            pltpu.VMEM((G, H), jnp.float32),
            pltpu.VMEM((G, 1), jnp.float32),
        ],
    )(p3, hs3, dinv, b3.reshape(1, H), batch_p, Wfc, bfc.reshape(1, C))

    return out


# speedup vs baseline: 1.0129x; 1.0129x over previous
import functools

import jax
import jax.numpy as jnp
from jax import lax
from jax.experimental import pallas as pl
from jax.experimental.pallas import tpu as pltpu
from jax.experimental.pallas import tpu_sc as plsc

_NC = 2
_NS = 16
_CHUNK = 128
_BR = 1024
_G = 64


def _sc_mesh():
    return plsc.VectorSubcoreMesh(core_axis_name="c", subcore_axis_name="s",
                                  num_cores=_NC, num_subcores=_NS)


def _make_deg_kernel(NP, NCH):
    rpt = NP // _NS

    @functools.partial(
        pl.kernel,
        out_type=jax.ShapeDtypeStruct((_NC, NP), jnp.float32),
        mesh=_sc_mesh(),
        compiler_params=pltpu.CompilerParams(use_tc_tiling_on_sc=False, skip_device_barrier=True, disable_bounds_checks=True, disable_semaphore_checks=True),
        scratch_types=[
            pltpu.VMEM((NCH, _CHUNK), jnp.int32),
            pltpu.VMEM((_CHUNK,), jnp.float32),
            pltpu.VMEM((rpt,), jnp.float32),
            pltpu.VMEM_SHARED((NP,), jnp.float32),
        ],
    )
    def deg_kernel(d_hbm, out_hbm, didx, ones, zbuf, acc):
        c = lax.axis_index("c")
        t = lax.axis_index("s")
        pltpu.sync_copy(d_hbm.at[c, t], didx)
        one16 = jnp.ones((16,), jnp.float32)
        zero16 = jnp.zeros((16,), jnp.float32)
        for i in range(_CHUNK // 16):
            ones[pl.ds(i * 16, 16)] = one16
        for i in range(rpt // 16):
            zbuf[pl.ds(i * 16, 16)] = zero16
        pltpu.sync_copy(zbuf, acc.at[pl.ds(t * rpt, rpt)])
        plsc.subcore_barrier()

        def body(j, carry):
            pltpu.sync_copy(ones, acc.at[didx.at[j]], add=True)
            return carry

        lax.fori_loop(0, NCH, body, 0, unroll=False)
        plsc.subcore_barrier()
        pltpu.sync_copy(acc.at[pl.ds(t * rpt, rpt)],
                        out_hbm.at[c, pl.ds(t * rpt, rpt)])

    return deg_kernel


def _make_agg_kernel(NP, NCH, H):
    rpt = NP // _NS

    @functools.partial(
        pl.kernel,
        out_type=jax.ShapeDtypeStruct((_NC, NP, H), jnp.float32),
        mesh=_sc_mesh(),
        compiler_params=pltpu.CompilerParams(use_tc_tiling_on_sc=False, skip_device_barrier=True, disable_bounds_checks=True, disable_semaphore_checks=True),
        scratch_types=[
            pltpu.VMEM((NCH, _CHUNK), jnp.int32),
            pltpu.VMEM((NCH, _CHUNK), jnp.int32),
            pltpu.VMEM((8, _CHUNK, H), jnp.float32),
            pltpu.VMEM_SHARED((NP, H), jnp.float32),
            [pltpu.SemaphoreType.DMA] * 8,
            [pltpu.SemaphoreType.DMA] * 8,
            pltpu.SemaphoreType.DMA,
        ],
    )
    def agg_kernel(hs_hbm, s_hbm, d_hbm, out_hbm, sidx, didx, rows, acc,
                   gsem, ssem, isem):
        c = lax.axis_index("c")
        t = lax.axis_index("s")

        def gather(j, b):
            return pltpu.make_async_copy(hs_hbm.at[sidx.at[j]], rows.at[b],
                                         gsem[b])

        def scatter_start(j, b):
            pltpu.async_copy(rows.at[b], acc.at[didx.at[j]], ssem[b],
                             add=True)

        def scatter_wait(j, b):
            pltpu.make_async_copy(rows.at[b], acc.at[didx.at[j]],
                                  ssem[b]).wait()

        pltpu.sync_copy(s_hbm.at[c, t], sidx)
        init_cp = pltpu.make_async_copy(hs_hbm.at[pl.ds(t * rpt, rpt)],
                                        acc.at[pl.ds(t * rpt, rpt)], isem)
        init_cp.start()
        for b in range(4):
            gather(b, b).start()
        pltpu.sync_copy(d_hbm.at[c, t], didx)
        init_cp.wait()
        plsc.subcore_barrier()

        def body(jj, carry):
            for b in range(8):
                j = 8 * jj + b
                gather(j, b).wait()
                scatter_start(j, b)
                b4 = (b + 4) % 8
                j4 = j + 4

                @pl.when(j4 >= 8)
                def _():
                    scatter_wait(j4 - 8, b4)

                @pl.when(j4 < NCH)
                def _():
                    gather(j4, b4).start()
            return carry

        lax.fori_loop(0, NCH // 8, body, 0, unroll=False)
        for k in range(4, 0, -1):
            scatter_wait(NCH - k, (NCH - k) % 8)
        plsc.subcore_barrier()
        pltpu.sync_copy(acc.at[pl.ds(t * rpt, rpt)],
                        out_hbm.at[c, pl.ds(t * rpt, rpt)])

    return agg_kernel


def _prep_body(x_ref, w_ref, degp_ref, hs_ref, dinv_ref, *, Nreal):
    i = pl.program_id(0)
    deg = (degp_ref[0:1, :] + degp_ref[1:2, :] + 1.0).reshape(_BR, 1)
    dinv = lax.rsqrt(deg)
    h = jnp.dot(x_ref[...], w_ref[...], preferred_element_type=jnp.float32)
    rows = i * _BR + lax.broadcasted_iota(jnp.int32, (_BR, 1), 0)
    valid = rows < Nreal
    hs_ref[...] = jnp.where(valid, dinv * h, 0.0)
    dinv_ref[...] = jnp.where(valid, dinv, 1.0)


def _mid_body(p_ref, hs_ref, dinv_ref, b_ref, w_ref, out_ref, *, Nreal):
    i = pl.program_id(0)
    z = dinv_ref[...] * (p_ref[0] + p_ref[1] - hs_ref[...]) + b_ref[...]
    h = jnp.maximum(z, 0.0)
    hn = jnp.dot(h, w_ref[...], preferred_element_type=jnp.float32)
    rows = i * _BR + lax.broadcasted_iota(jnp.int32, (_BR, 1), 0)
    out_ref[...] = jnp.where(rows < Nreal, dinv_ref[...] * hn, 0.0)


def _final_body(p_ref, hs_ref, dinv_ref, b_ref, batch_ref, wfc_ref, bfc_ref,
                out_ref, pooled_acc, cnt_acc, *, nblocks, G):
    i = pl.program_id(0)

    @pl.when(i == 0)
    def _():
        pooled_acc[...] = jnp.zeros_like(pooled_acc)
        cnt_acc[...] = jnp.zeros_like(cnt_acc)

    z = dinv_ref[...] * (p_ref[0] + p_ref[1] - hs_ref[...]) + b_ref[...]
    gids = lax.broadcasted_iota(jnp.int32, (G, _BR), 0)
    mask = (gids == batch_ref[...]).astype(jnp.float32)
    pooled_acc[...] += jnp.dot(mask, z, preferred_element_type=jnp.float32)
    cnt_acc[...] += jnp.sum(mask, axis=1, keepdims=True)

    @pl.when(i == nblocks - 1)
    def _():
        pooled = pooled_acc[...] / jnp.maximum(cnt_acc[...], 1.0)
        out_ref[...] = (jnp.dot(pooled, wfc_ref[...],
                                preferred_element_type=jnp.float32)
                        + bfc_ref[...])


def kernel(x, edge_index, batch, W1, b1, W2, b2, W3, b3, Wfc, bfc):
    N, D = x.shape
    E = edge_index.shape[1]
    H = W1.shape[1]
    C = Wfc.shape[1]
    G = _G

    NP = ((N + 1 + _BR - 1) // _BR) * _BR
    nblocks = NP // _BR
    per_round = _NC * _NS * _CHUNK
    NCH = -(-E // per_round)
    NCH = ((NCH + 7) // 8) * 8
    EP = NCH * per_round
    pad = EP - E
    npad_rows = NP - N

    spread = (jnp.arange(pad, dtype=jnp.int32) % npad_rows) + N
    s_arr = jnp.concatenate([edge_index[0], spread]).reshape(
        _NC, _NS, NCH, _CHUNK)
    d_arr = jnp.concatenate([edge_index[1], spread]).reshape(
        _NC, _NS, NCH, _CHUNK)
    batch_p = jnp.concatenate(
        [batch, jnp.full((NP - N,), G, jnp.int32)]).reshape(1, NP)

    deg_kernel = _make_deg_kernel(NP, NCH)
    agg_kernel = _make_agg_kernel(NP, NCH, H)

    degp = deg_kernel(d_arr)

    hs1, dinv = pl.pallas_call(
        functools.partial(_prep_body, Nreal=N),
        grid=(nblocks,),
        in_specs=[
            pl.BlockSpec((_BR, D), lambda i: (i, 0)),
            pl.BlockSpec((D, H), lambda i: (0, 0)),
            pl.BlockSpec((2, _BR), lambda i: (0, i)),
        ],
        out_specs=[
            pl.BlockSpec((_BR, H), lambda i: (i, 0)),
            pl.BlockSpec((_BR, 1), lambda i: (i, 0)),
        ],
        out_shape=[
            jax.ShapeDtypeStruct((NP, H), jnp.float32),
            jax.ShapeDtypeStruct((NP, 1), jnp.float32),
        ],
    )(x, W1, degp)

    def mid(p, hs, b, W):
        return pl.pallas_call(
            functools.partial(_mid_body, Nreal=N),
            grid=(nblocks,),
            in_specs=[
                pl.BlockSpec((_NC, _BR, H), lambda i: (0, i, 0)),
                pl.BlockSpec((_BR, H), lambda i: (i, 0)),
                pl.BlockSpec((_BR, 1), lambda i: (i, 0)),
                pl.BlockSpec((1, H), lambda i: (0, 0)),
                pl.BlockSpec((H, H), lambda i: (0, 0)),
            ],
            out_specs=pl.BlockSpec((_BR, H), lambda i: (i, 0)),
            out_shape=jax.ShapeDtypeStruct((NP, H), jnp.float32),
        )(p, hs, dinv, b.reshape(1, H), W)

    p1 = agg_kernel(hs1, s_arr, d_arr)
    hs2 = mid(p1, hs1, b1, W2)
    p2 = agg_kernel(hs2, s_arr, d_arr)
    hs3 = mid(p2, hs2, b2, W3)
    p3 = agg_kernel(hs3, s_arr, d_arr)

    out = pl.pallas_call(
        functools.partial(_final_body, nblocks=nblocks, G=G),
        grid=(nblocks,),
        in_specs=[
            pl.BlockSpec((_NC, _BR, H), lambda i: (0, i, 0)),
            pl.BlockSpec((_BR, H), lambda i: (i, 0)),
            pl.BlockSpec((_BR, 1), lambda i: (i, 0)),
            pl.BlockSpec((1, H), lambda i: (0, 0)),
            pl.BlockSpec((1, _BR), lambda i: (0, i)),
            pl.BlockSpec((H, C), lambda i: (0, 0)),
            pl.BlockSpec((1, C), lambda i: (0, 0)),
        ],
        out_specs=pl.BlockSpec((G, C), lambda i: (0, 0)),
        out_shape=jax.ShapeDtypeStruct((G, C), jnp.float32),
        scratch_shapes=[
            pltpu.VMEM((G, H), jnp.float32),
            pltpu.VMEM((G, 1), jnp.float32),
        ],
    )(p3, hs3, dinv, b3.reshape(1, H), batch_p, Wfc, bfc.reshape(1, C))

    return out


# speedup vs baseline: 1.2758x; 1.2595x over previous
import functools

import jax
import jax.numpy as jnp
from jax import lax
from jax.experimental import pallas as pl
from jax.experimental.pallas import tpu as pltpu
from jax.experimental.pallas import tpu_sc as plsc

_NC = 2
_NS = 16
_CHUNK = 128
_BR = 1024
_G = 64


def _sc_mesh():
    return plsc.VectorSubcoreMesh(core_axis_name="c", subcore_axis_name="s",
                                  num_cores=_NC, num_subcores=_NS)


def _make_deg_kernel(NP, NCH):
    rpt = NP // _NS

    @functools.partial(
        pl.kernel,
        out_type=jax.ShapeDtypeStruct((_NC, NP), jnp.float32),
        mesh=_sc_mesh(),
        compiler_params=pltpu.CompilerParams(use_tc_tiling_on_sc=False, skip_device_barrier=True, disable_bounds_checks=True, disable_semaphore_checks=True),
        scratch_types=[
            pltpu.VMEM((NCH, _CHUNK), jnp.int32),
            pltpu.VMEM((_CHUNK,), jnp.float32),
            pltpu.VMEM((rpt,), jnp.float32),
            pltpu.VMEM_SHARED((NP,), jnp.float32),
        ],
    )
    def deg_kernel(d_hbm, out_hbm, didx, ones, zbuf, acc):
        c = lax.axis_index("c")
        t = lax.axis_index("s")
        pltpu.sync_copy(d_hbm.at[c, t], didx)
        one16 = jnp.ones((16,), jnp.float32)
        zero16 = jnp.zeros((16,), jnp.float32)
        for i in range(_CHUNK // 16):
            ones[pl.ds(i * 16, 16)] = one16
        for i in range(rpt // 16):
            zbuf[pl.ds(i * 16, 16)] = zero16
        pltpu.sync_copy(zbuf, acc.at[pl.ds(t * rpt, rpt)])
        plsc.subcore_barrier()

        def body(j, carry):
            pltpu.sync_copy(ones, acc.at[didx.at[j]], add=True)
            return carry

        lax.fori_loop(0, NCH, body, 0, unroll=False)
        plsc.subcore_barrier()
        pltpu.sync_copy(acc.at[pl.ds(t * rpt, rpt)],
                        out_hbm.at[c, pl.ds(t * rpt, rpt)])

    return deg_kernel


def _make_agg_kernel(NP, NCH, H):
    rpt = NP // _NS

    @functools.partial(
        pl.kernel,
        out_type=jax.ShapeDtypeStruct((_NC, NP, H), jnp.bfloat16),
        mesh=_sc_mesh(),
        compiler_params=pltpu.CompilerParams(use_tc_tiling_on_sc=False, skip_device_barrier=True, disable_bounds_checks=True, disable_semaphore_checks=True),
        scratch_types=[
            pltpu.VMEM((NCH, _CHUNK), jnp.int32),
            pltpu.VMEM((NCH, _CHUNK), jnp.int32),
            pltpu.VMEM((8, _CHUNK, H), jnp.bfloat16),
            pltpu.VMEM_SHARED((NP, H), jnp.bfloat16),
            [pltpu.SemaphoreType.DMA] * 8,
            [pltpu.SemaphoreType.DMA] * 8,
            pltpu.SemaphoreType.DMA,
        ],
    )
    def agg_kernel(hs_hbm, s_hbm, d_hbm, out_hbm, sidx, didx, rows, acc,
                   gsem, ssem, isem):
        c = lax.axis_index("c")
        t = lax.axis_index("s")

        def gather(j, b):
            return pltpu.make_async_copy(hs_hbm.at[sidx.at[j]], rows.at[b],
                                         gsem[b])

        def scatter_start(j, b):
            pltpu.async_copy(rows.at[b], acc.at[didx.at[j]], ssem[b],
                             add=True)

        def scatter_wait(j, b):
            pltpu.make_async_copy(rows.at[b], acc.at[didx.at[j]],
                                  ssem[b]).wait()

        pltpu.sync_copy(s_hbm.at[c, t], sidx)
        init_cp = pltpu.make_async_copy(hs_hbm.at[pl.ds(t * rpt, rpt)],
                                        acc.at[pl.ds(t * rpt, rpt)], isem)
        init_cp.start()
        for b in range(4):
            gather(b, b).start()
        pltpu.sync_copy(d_hbm.at[c, t], didx)
        init_cp.wait()
        plsc.subcore_barrier()

        def body(jj, carry):
            for b in range(8):
                j = 8 * jj + b
                gather(j, b).wait()
                scatter_start(j, b)
                b4 = (b + 4) % 8
                j4 = j + 4

                @pl.when(j4 >= 8)
                def _():
                    scatter_wait(j4 - 8, b4)

                @pl.when(j4 < NCH)
                def _():
                    gather(j4, b4).start()
            return carry

        lax.fori_loop(0, NCH // 8, body, 0, unroll=False)
        for k in range(4, 0, -1):
            scatter_wait(NCH - k, (NCH - k) % 8)
        plsc.subcore_barrier()
        pltpu.sync_copy(acc.at[pl.ds(t * rpt, rpt)],
                        out_hbm.at[c, pl.ds(t * rpt, rpt)])

    return agg_kernel


def _prep_body(x_ref, w_ref, degp_ref, hs_ref, dinv_ref, *, Nreal):
    i = pl.program_id(0)
    deg = (degp_ref[0:1, :] + degp_ref[1:2, :] + 1.0).reshape(_BR, 1)
    dinv = lax.rsqrt(deg)
    h = jnp.dot(x_ref[...], w_ref[...], preferred_element_type=jnp.float32)
    rows = i * _BR + lax.broadcasted_iota(jnp.int32, (_BR, 1), 0)
    valid = rows < Nreal
    hs_ref[...] = jnp.where(valid, dinv * h, 0.0).astype(jnp.bfloat16)
    dinv_ref[...] = jnp.where(valid, dinv, 1.0)


def _mid_body(p_ref, hs_ref, dinv_ref, b_ref, w_ref, out_ref, *, Nreal):
    i = pl.program_id(0)
    psum = (p_ref[0].astype(jnp.float32) + p_ref[1].astype(jnp.float32)
            - hs_ref[...].astype(jnp.float32))
    z = dinv_ref[...] * psum + b_ref[...]
    h = jnp.maximum(z, 0.0)
    hn = jnp.dot(h, w_ref[...], preferred_element_type=jnp.float32)
    rows = i * _BR + lax.broadcasted_iota(jnp.int32, (_BR, 1), 0)
    out_ref[...] = jnp.where(rows < Nreal, dinv_ref[...] * hn,
                             0.0).astype(jnp.bfloat16)


def _final_body(p_ref, hs_ref, dinv_ref, b_ref, batch_ref, wfc_ref, bfc_ref,
                out_ref, pooled_acc, cnt_acc, *, nblocks, G):
    i = pl.program_id(0)

    @pl.when(i == 0)
    def _():
        pooled_acc[...] = jnp.zeros_like(pooled_acc)
        cnt_acc[...] = jnp.zeros_like(cnt_acc)

    psum = (p_ref[0].astype(jnp.float32) + p_ref[1].astype(jnp.float32)
            - hs_ref[...].astype(jnp.float32))
    z = dinv_ref[...] * psum + b_ref[...]
    gids = lax.broadcasted_iota(jnp.int32, (G, _BR), 0)
    mask = (gids == batch_ref[...]).astype(jnp.float32)
    pooled_acc[...] += jnp.dot(mask, z, preferred_element_type=jnp.float32)
    cnt_acc[...] += jnp.sum(mask, axis=1, keepdims=True)

    @pl.when(i == nblocks - 1)
    def _():
        pooled = pooled_acc[...] / jnp.maximum(cnt_acc[...], 1.0)
        out_ref[...] = (jnp.dot(pooled, wfc_ref[...],
                                preferred_element_type=jnp.float32)
                        + bfc_ref[...])


def kernel(x, edge_index, batch, W1, b1, W2, b2, W3, b3, Wfc, bfc):
    N, D = x.shape
    E = edge_index.shape[1]
    H = W1.shape[1]
    C = Wfc.shape[1]
    G = _G

    NP = ((N + 1 + _BR - 1) // _BR) * _BR
    nblocks = NP // _BR
    per_round = _NC * _NS * _CHUNK
    NCH = -(-E // per_round)
    NCH = ((NCH + 7) // 8) * 8
    EP = NCH * per_round
    pad = EP - E
    npad_rows = NP - N

    spread = (jnp.arange(pad, dtype=jnp.int32) % npad_rows) + N
    s_arr = jnp.concatenate([edge_index[0], spread]).reshape(
        _NC, _NS, NCH, _CHUNK)
    d_arr = jnp.concatenate([edge_index[1], spread]).reshape(
        _NC, _NS, NCH, _CHUNK)
    batch_p = jnp.concatenate(
        [batch, jnp.full((NP - N,), G, jnp.int32)]).reshape(1, NP)

    deg_kernel = _make_deg_kernel(NP, NCH)
    agg_kernel = _make_agg_kernel(NP, NCH, H)

    degp = deg_kernel(d_arr)

    hs1, dinv = pl.pallas_call(
        functools.partial(_prep_body, Nreal=N),
        grid=(nblocks,),
        in_specs=[
            pl.BlockSpec((_BR, D), lambda i: (i, 0)),
            pl.BlockSpec((D, H), lambda i: (0, 0)),
            pl.BlockSpec((2, _BR), lambda i: (0, i)),
        ],
        out_specs=[
            pl.BlockSpec((_BR, H), lambda i: (i, 0)),
            pl.BlockSpec((_BR, 1), lambda i: (i, 0)),
        ],
        out_shape=[
            jax.ShapeDtypeStruct((NP, H), jnp.bfloat16),
            jax.ShapeDtypeStruct((NP, 1), jnp.float32),
        ],
    )(x, W1, degp)

    def mid(p, hs, b, W):
        return pl.pallas_call(
            functools.partial(_mid_body, Nreal=N),
            grid=(nblocks,),
            in_specs=[
                pl.BlockSpec((_NC, _BR, H), lambda i: (0, i, 0)),
                pl.BlockSpec((_BR, H), lambda i: (i, 0)),
                pl.BlockSpec((_BR, 1), lambda i: (i, 0)),
                pl.BlockSpec((1, H), lambda i: (0, 0)),
                pl.BlockSpec((H, H), lambda i: (0, 0)),
            ],
            out_specs=pl.BlockSpec((_BR, H), lambda i: (i, 0)),
            out_shape=jax.ShapeDtypeStruct((NP, H), jnp.bfloat16),
        )(p, hs, dinv, b.reshape(1, H), W)

    p1 = agg_kernel(hs1, s_arr, d_arr)
    hs2 = mid(p1, hs1, b1, W2)
    p2 = agg_kernel(hs2, s_arr, d_arr)
    hs3 = mid(p2, hs2, b2, W3)
    p3 = agg_kernel(hs3, s_arr, d_arr)

    out = pl.pallas_call(
        functools.partial(_final_body, nblocks=nblocks, G=G),
        grid=(nblocks,),
        in_specs=[
            pl.BlockSpec((_NC, _BR, H), lambda i: (0, i, 0)),
            pl.BlockSpec((_BR, H), lambda i: (i, 0)),
            pl.BlockSpec((_BR, 1), lambda i: (i, 0)),
            pl.BlockSpec((1, H), lambda i: (0, 0)),
            pl.BlockSpec((1, _BR), lambda i: (0, i)),
            pl.BlockSpec((H, C), lambda i: (0, 0)),
            pl.BlockSpec((1, C), lambda i: (0, 0)),
        ],
        out_specs=pl.BlockSpec((G, C), lambda i: (0, 0)),
        out_shape=jax.ShapeDtypeStruct((G, C), jnp.float32),
        scratch_shapes=[
            pltpu.VMEM((G, H), jnp.float32),
            pltpu.VMEM((G, 1), jnp.float32),
        ],
    )(p3, hs3, dinv, b3.reshape(1, H), batch_p, Wfc, bfc.reshape(1, C))

    return out


# speedup vs baseline: 1.3566x; 1.0634x over previous
import functools

import jax
import jax.numpy as jnp
from jax import lax
from jax.experimental import pallas as pl
from jax.experimental.pallas import tpu as pltpu
from jax.experimental.pallas import tpu_sc as plsc

_NC = 2
_NS = 16
_CHUNK = 128
_BR = 2048
_G = 64


def _sc_mesh():
    return plsc.VectorSubcoreMesh(core_axis_name="c", subcore_axis_name="s",
                                  num_cores=_NC, num_subcores=_NS)


def _make_deg_kernel(NP, NCH):
    rpt = NP // _NS

    @functools.partial(
        pl.kernel,
        out_type=jax.ShapeDtypeStruct((_NC, NP), jnp.float32),
        mesh=_sc_mesh(),
        compiler_params=pltpu.CompilerParams(use_tc_tiling_on_sc=False, skip_device_barrier=True, disable_bounds_checks=True, disable_semaphore_checks=True),
        scratch_types=[
            pltpu.VMEM((NCH, _CHUNK), jnp.int32),
            pltpu.VMEM((_CHUNK,), jnp.float32),
            pltpu.VMEM((rpt,), jnp.float32),
            pltpu.VMEM_SHARED((NP,), jnp.float32),
            [pltpu.SemaphoreType.DMA] * 4,
        ],
    )
    def deg_kernel(d_hbm, out_hbm, didx, ones, zbuf, acc, dsem):
        c = lax.axis_index("c")
        t = lax.axis_index("s")
        pltpu.sync_copy(d_hbm.at[c, t], didx)
        one16 = jnp.ones((16,), jnp.float32)
        zero16 = jnp.zeros((16,), jnp.float32)
        for i in range(_CHUNK // 16):
            ones[pl.ds(i * 16, 16)] = one16
        for i in range(rpt // 16):
            zbuf[pl.ds(i * 16, 16)] = zero16
        pltpu.sync_copy(zbuf, acc.at[pl.ds(t * rpt, rpt)])
        plsc.subcore_barrier()

        def body(jj, carry):
            for b in range(4):
                j = 4 * jj + b
                pltpu.async_copy(ones, acc.at[didx.at[j]], dsem[b],
                                 add=True)

                @pl.when(j >= 4)
                def _():
                    pltpu.make_async_copy(ones, acc.at[didx.at[j - 4]],
                                          dsem[b]).wait()
            return carry

        lax.fori_loop(0, NCH // 4, body, 0, unroll=False)
        for k in range(4, 0, -1):
            pltpu.make_async_copy(ones, acc.at[didx.at[NCH - k]],
                                  dsem[(NCH - k) % 4]).wait()
        plsc.subcore_barrier()
        pltpu.sync_copy(acc.at[pl.ds(t * rpt, rpt)],
                        out_hbm.at[c, pl.ds(t * rpt, rpt)])

    return deg_kernel


def _make_agg_kernel(NP, NCH, H):
    rpt = NP // _NS

    @functools.partial(
        pl.kernel,
        out_type=jax.ShapeDtypeStruct((_NC, NP, H), jnp.bfloat16),
        mesh=_sc_mesh(),
        compiler_params=pltpu.CompilerParams(use_tc_tiling_on_sc=False, skip_device_barrier=True, disable_bounds_checks=True, disable_semaphore_checks=True),
        scratch_types=[
            pltpu.VMEM((NCH, _CHUNK), jnp.int32),
            pltpu.VMEM((NCH, _CHUNK), jnp.int32),
            pltpu.VMEM((8, _CHUNK, H), jnp.bfloat16),
            pltpu.VMEM_SHARED((NP, H), jnp.bfloat16),
            [pltpu.SemaphoreType.DMA] * 8,
            [pltpu.SemaphoreType.DMA] * 8,
            pltpu.SemaphoreType.DMA,
        ],
    )
    def agg_kernel(hs_hbm, s_hbm, d_hbm, out_hbm, sidx, didx, rows, acc,
                   gsem, ssem, isem):
        c = lax.axis_index("c")
        t = lax.axis_index("s")

        def gather(j, b):
            return pltpu.make_async_copy(hs_hbm.at[sidx.at[j]], rows.at[b],
                                         gsem[b])

        def scatter_start(j, b):
            pltpu.async_copy(rows.at[b], acc.at[didx.at[j]], ssem[b],
                             add=True)

        def scatter_wait(j, b):
            pltpu.make_async_copy(rows.at[b], acc.at[didx.at[j]],
                                  ssem[b]).wait()

        pltpu.sync_copy(s_hbm.at[c, t], sidx)
        init_cp = pltpu.make_async_copy(hs_hbm.at[pl.ds(t * rpt, rpt)],
                                        acc.at[pl.ds(t * rpt, rpt)], isem)
        init_cp.start()
        for b in range(4):
            gather(b, b).start()
        pltpu.sync_copy(d_hbm.at[c, t], didx)
        init_cp.wait()
        plsc.subcore_barrier()

        def body(jj, carry):
            for b in range(8):
                j = 8 * jj + b
                gather(j, b).wait()
                scatter_start(j, b)
                b4 = (b + 4) % 8
                j4 = j + 4

                @pl.when(j4 >= 8)
                def _():
                    scatter_wait(j4 - 8, b4)

                @pl.when(j4 < NCH)
                def _():
                    gather(j4, b4).start()
            return carry

        lax.fori_loop(0, NCH // 8, body, 0, unroll=False)
        for k in range(4, 0, -1):
            scatter_wait(NCH - k, (NCH - k) % 8)
        plsc.subcore_barrier()
        pltpu.sync_copy(acc.at[pl.ds(t * rpt, rpt)],
                        out_hbm.at[c, pl.ds(t * rpt, rpt)])

    return agg_kernel


def _prep_body(x_ref, w_ref, degp_ref, hs_ref, dinv_ref, *, Nreal):
    i = pl.program_id(0)
    deg = (degp_ref[0:1, :] + degp_ref[1:2, :] + 1.0).reshape(_BR, 1)
    dinv = lax.rsqrt(deg)
    h = jnp.dot(x_ref[...], w_ref[...], preferred_element_type=jnp.float32)
    rows = i * _BR + lax.broadcasted_iota(jnp.int32, (_BR, 1), 0)
    valid = rows < Nreal
    hs_ref[...] = jnp.where(valid, dinv * h, 0.0).astype(jnp.bfloat16)
    dinv_ref[...] = jnp.where(valid, dinv, 1.0)


def _mid_body(p_ref, hs_ref, dinv_ref, b_ref, w_ref, out_ref, *, Nreal):
    i = pl.program_id(0)
    psum = (p_ref[0].astype(jnp.float32) + p_ref[1].astype(jnp.float32)
            - hs_ref[...].astype(jnp.float32))
    z = dinv_ref[...] * psum + b_ref[...]
    h = jnp.maximum(z, 0.0)
    hn = jnp.dot(h, w_ref[...], preferred_element_type=jnp.float32)
    rows = i * _BR + lax.broadcasted_iota(jnp.int32, (_BR, 1), 0)
    out_ref[...] = jnp.where(rows < Nreal, dinv_ref[...] * hn,
                             0.0).astype(jnp.bfloat16)


def _final_body(p_ref, hs_ref, dinv_ref, b_ref, batch_ref, wfc_ref, bfc_ref,
                out_ref, pooled_acc, cnt_acc, *, nblocks, G):
    i = pl.program_id(0)

    @pl.when(i == 0)
    def _():
        pooled_acc[...] = jnp.zeros_like(pooled_acc)
        cnt_acc[...] = jnp.zeros_like(cnt_acc)

    psum = (p_ref[0].astype(jnp.float32) + p_ref[1].astype(jnp.float32)
            - hs_ref[...].astype(jnp.float32))
    z = dinv_ref[...] * psum + b_ref[...]
    gids = lax.broadcasted_iota(jnp.int32, (G, _BR), 0)
    mask = (gids == batch_ref[...]).astype(jnp.float32)
    pooled_acc[...] += jnp.dot(mask, z, preferred_element_type=jnp.float32)
    cnt_acc[...] += jnp.sum(mask, axis=1, keepdims=True)

    @pl.when(i == nblocks - 1)
    def _():
        pooled = pooled_acc[...] / jnp.maximum(cnt_acc[...], 1.0)
        out_ref[...] = (jnp.dot(pooled, wfc_ref[...],
                                preferred_element_type=jnp.float32)
                        + bfc_ref[...])


def kernel(x, edge_index, batch, W1, b1, W2, b2, W3, b3, Wfc, bfc):
    N, D = x.shape
    E = edge_index.shape[1]
    H = W1.shape[1]
    C = Wfc.shape[1]
    G = _G

    NP = ((N + 1 + _BR - 1) // _BR) * _BR
    nblocks = NP // _BR
    per_round = _NC * _NS * _CHUNK
    NCH = -(-E // per_round)
    NCH = ((NCH + 7) // 8) * 8
    EP = NCH * per_round
    pad = EP - E
    npad_rows = NP - N

    spread = (jnp.arange(pad, dtype=jnp.int32) % npad_rows) + N
    s_arr = jnp.concatenate([edge_index[0], spread]).reshape(
        _NC, _NS, NCH, _CHUNK)
    d_arr = jnp.concatenate([edge_index[1], spread]).reshape(
        _NC, _NS, NCH, _CHUNK)
    batch_p = jnp.concatenate(
        [batch, jnp.full((NP - N,), G, jnp.int32)]).reshape(1, NP)

    deg_kernel = _make_deg_kernel(NP, NCH)
    agg_kernel = _make_agg_kernel(NP, NCH, H)

    degp = deg_kernel(d_arr)

    hs1, dinv = pl.pallas_call(
        functools.partial(_prep_body, Nreal=N),
        grid=(nblocks,),
        in_specs=[
            pl.BlockSpec((_BR, D), lambda i: (i, 0)),
            pl.BlockSpec((D, H), lambda i: (0, 0)),
            pl.BlockSpec((2, _BR), lambda i: (0, i)),
        ],
        out_specs=[
            pl.BlockSpec((_BR, H), lambda i: (i, 0)),
            pl.BlockSpec((_BR, 1), lambda i: (i, 0)),
        ],
        out_shape=[
            jax.ShapeDtypeStruct((NP, H), jnp.bfloat16),
            jax.ShapeDtypeStruct((NP, 1), jnp.float32),
        ],
    )(x, W1, degp)

    def mid(p, hs, b, W):
        return pl.pallas_call(
            functools.partial(_mid_body, Nreal=N),
            grid=(nblocks,),
            in_specs=[
                pl.BlockSpec((_NC, _BR, H), lambda i: (0, i, 0)),
                pl.BlockSpec((_BR, H), lambda i: (i, 0)),
                pl.BlockSpec((_BR, 1), lambda i: (i, 0)),
                pl.BlockSpec((1, H), lambda i: (0, 0)),
                pl.BlockSpec((H, H), lambda i: (0, 0)),
            ],
            out_specs=pl.BlockSpec((_BR, H), lambda i: (i, 0)),
            out_shape=jax.ShapeDtypeStruct((NP, H), jnp.bfloat16),
        )(p, hs, dinv, b.reshape(1, H), W)

    p1 = agg_kernel(hs1, s_arr, d_arr)
    hs2 = mid(p1, hs1, b1, W2)
    p2 = agg_kernel(hs2, s_arr, d_arr)
    hs3 = mid(p2, hs2, b2, W3)
    p3 = agg_kernel(hs3, s_arr, d_arr)

    out = pl.pallas_call(
        functools.partial(_final_body, nblocks=nblocks, G=G),
        grid=(nblocks,),
        in_specs=[
            pl.BlockSpec((_NC, _BR, H), lambda i: (0, i, 0)),
            pl.BlockSpec((_BR, H), lambda i: (i, 0)),
            pl.BlockSpec((_BR, 1), lambda i: (i, 0)),
            pl.BlockSpec((1, H), lambda i: (0, 0)),
            pl.BlockSpec((1, _BR), lambda i: (0, i)),
            pl.BlockSpec((H, C), lambda i: (0, 0)),
            pl.BlockSpec((1, C), lambda i: (0, 0)),
        ],
        out_specs=pl.BlockSpec((G, C), lambda i: (0, 0)),
        out_shape=jax.ShapeDtypeStruct((G, C), jnp.float32),
        scratch_shapes=[
            pltpu.VMEM((G, H), jnp.float32),
            pltpu.VMEM((G, 1), jnp.float32),
        ],
    )(p3, hs3, dinv, b3.reshape(1, H), batch_p, Wfc, bfc.reshape(1, C))

    return out


# speedup vs baseline: 1.3759x; 1.0142x over previous
import functools

import jax
import jax.numpy as jnp
from jax import lax
from jax.experimental import pallas as pl
from jax.experimental.pallas import tpu as pltpu
from jax.experimental.pallas import tpu_sc as plsc

_NC = 2
_NS = 16
_CHUNK = 128
_BR = 2048
_G = 64


def _sc_mesh():
    return plsc.VectorSubcoreMesh(core_axis_name="c", subcore_axis_name="s",
                                  num_cores=_NC, num_subcores=_NS)


def _make_deg_kernel(NP, NCH):
    rpt = NP // _NS

    @functools.partial(
        pl.kernel,
        out_type=jax.ShapeDtypeStruct((_NC, NP), jnp.float32),
        mesh=_sc_mesh(),
        compiler_params=pltpu.CompilerParams(use_tc_tiling_on_sc=False, skip_device_barrier=True, disable_bounds_checks=True, disable_semaphore_checks=True),
        scratch_types=[
            pltpu.VMEM((NCH, _CHUNK), jnp.int32),
            pltpu.VMEM((_CHUNK,), jnp.float32),
            pltpu.VMEM((rpt,), jnp.float32),
            pltpu.VMEM_SHARED((NP,), jnp.float32),
            [pltpu.SemaphoreType.DMA] * 4,
        ],
    )
    def deg_kernel(d_hbm, out_hbm, didx, ones, zbuf, acc, dsem):
        c = lax.axis_index("c")
        t = lax.axis_index("s")
        pltpu.sync_copy(d_hbm.at[c, t], didx)
        one16 = jnp.ones((16,), jnp.float32)
        zero16 = jnp.zeros((16,), jnp.float32)
        for i in range(_CHUNK // 16):
            ones[pl.ds(i * 16, 16)] = one16
        for i in range(rpt // 16):
            zbuf[pl.ds(i * 16, 16)] = zero16
        pltpu.sync_copy(zbuf, acc.at[pl.ds(t * rpt, rpt)])
        plsc.subcore_barrier()

        def body(jj, carry):
            for b in range(4):
                j = 4 * jj + b
                pltpu.async_copy(ones, acc.at[didx.at[j]], dsem[b],
                                 add=True)

                @pl.when(j >= 4)
                def _():
                    pltpu.make_async_copy(ones, acc.at[didx.at[j - 4]],
                                          dsem[b]).wait()
            return carry

        lax.fori_loop(0, NCH // 4, body, 0, unroll=False)
        for k in range(4, 0, -1):
            pltpu.make_async_copy(ones, acc.at[didx.at[NCH - k]],
                                  dsem[(NCH - k) % 4]).wait()
        plsc.subcore_barrier()
        pltpu.sync_copy(acc.at[pl.ds(t * rpt, rpt)],
                        out_hbm.at[c, pl.ds(t * rpt, rpt)])

    return deg_kernel


def _make_agg_kernel(NP, NCH, H):
    rpt = NP // _NS

    @functools.partial(
        pl.kernel,
        out_type=jax.ShapeDtypeStruct((_NC, NP, H), jnp.bfloat16),
        mesh=_sc_mesh(),
        compiler_params=pltpu.CompilerParams(use_tc_tiling_on_sc=False, skip_device_barrier=True, disable_bounds_checks=True, disable_semaphore_checks=True),
        scratch_types=[
            pltpu.VMEM((NCH, _CHUNK), jnp.int32),
            pltpu.VMEM((NCH, _CHUNK), jnp.int32),
            pltpu.VMEM((8, _CHUNK, H), jnp.bfloat16),
            pltpu.VMEM_SHARED((NP, H), jnp.bfloat16),
            [pltpu.SemaphoreType.DMA] * 8,
            [pltpu.SemaphoreType.DMA] * 8,
            pltpu.SemaphoreType.DMA,
        ],
    )
    def agg_kernel(hs_hbm, s_hbm, d_hbm, out_hbm, sidx, didx, rows, acc,
                   gsem, ssem, isem):
        c = lax.axis_index("c")
        t = lax.axis_index("s")

        def gather(j, b):
            return pltpu.make_async_copy(hs_hbm.at[sidx.at[j]], rows.at[b],
                                         gsem[b])

        def scatter_start(j, b):
            pltpu.async_copy(rows.at[b], acc.at[didx.at[j]], ssem[b],
                             add=True)

        def scatter_wait(j, b):
            pltpu.make_async_copy(rows.at[b], acc.at[didx.at[j]],
                                  ssem[b]).wait()

        pltpu.sync_copy(s_hbm.at[c, t], sidx)
        init_cp = pltpu.make_async_copy(hs_hbm.at[pl.ds(t * rpt, rpt)],
                                        acc.at[pl.ds(t * rpt, rpt)], isem)
        init_cp.start()
        for b in range(4):
            gather(b, b).start()
        pltpu.sync_copy(d_hbm.at[c, t], didx)
        init_cp.wait()
        plsc.subcore_barrier()

        def body(jj, carry):
            for b in range(8):
                j = 8 * jj + b
                gather(j, b).wait()
                scatter_start(j, b)
                b4 = (b + 4) % 8
                j4 = j + 4

                @pl.when(j4 >= 8)
                def _():
                    scatter_wait(j4 - 8, b4)

                @pl.when(j4 < NCH)
                def _():
                    gather(j4, b4).start()
            return carry

        lax.fori_loop(0, NCH // 8, body, 0, unroll=False)
        for k in range(4, 0, -1):
            scatter_wait(NCH - k, (NCH - k) % 8)
        plsc.subcore_barrier()
        pltpu.sync_copy(acc.at[pl.ds(t * rpt, rpt)],
                        out_hbm.at[c, pl.ds(t * rpt, rpt)])

    return agg_kernel


def _prep_body(x_ref, w_ref, degp_ref, hs_ref, dinv_ref, *, Nreal):
    i = pl.program_id(0)
    deg = (degp_ref[0:1, :] + degp_ref[1:2, :] + 1.0).reshape(_BR, 1)
    dinv = lax.rsqrt(deg)
    h = jnp.dot(x_ref[...], w_ref[...], preferred_element_type=jnp.float32)
    rows = i * _BR + lax.broadcasted_iota(jnp.int32, (_BR, 1), 0)
    valid = rows < Nreal
    hs_ref[...] = jnp.where(valid, dinv * h, 0.0).astype(jnp.bfloat16)
    dinv_ref[...] = jnp.where(valid, dinv, 1.0)


def _mid_body(p_ref, hs_ref, dinv_ref, b_ref, w_ref, out_ref, *, Nreal):
    i = pl.program_id(0)
    psum = (p_ref[0].astype(jnp.float32) + p_ref[1].astype(jnp.float32)
            - hs_ref[...].astype(jnp.float32))
    z = dinv_ref[...] * psum + b_ref[...]
    h = jnp.maximum(z, 0.0)
    hn = jnp.dot(h, w_ref[...], preferred_element_type=jnp.float32)
    rows = i * _BR + lax.broadcasted_iota(jnp.int32, (_BR, 1), 0)
    out_ref[...] = jnp.where(rows < Nreal, dinv_ref[...] * hn,
                             0.0).astype(jnp.bfloat16)


def _final_body(p_ref, hs_ref, dinv_ref, b_ref, batch_ref, wfc_ref, bfc_ref,
                out_ref, pooled_acc, cnt_acc, *, nblocks, G, Nreal):
    i = pl.program_id(0)

    @pl.when(i == 0)
    def _():
        pooled_acc[...] = jnp.zeros_like(pooled_acc)
        cnt_acc[...] = jnp.zeros_like(cnt_acc)

    psum = (p_ref[0].astype(jnp.float32) + p_ref[1].astype(jnp.float32)
            - hs_ref[...].astype(jnp.float32))
    z = dinv_ref[...] * psum + b_ref[...]
    gids = lax.broadcasted_iota(jnp.int32, (G, _BR), 0)
    cols = i * _BR + lax.broadcasted_iota(jnp.int32, (G, _BR), 1)
    mask = ((gids == batch_ref[...]) & (cols < Nreal)
            ).astype(jnp.float32)
    pooled_acc[...] += jnp.dot(mask, z, preferred_element_type=jnp.float32)
    cnt_acc[...] += jnp.sum(mask, axis=1, keepdims=True)

    @pl.when(i == nblocks - 1)
    def _():
        pooled = pooled_acc[...] / jnp.maximum(cnt_acc[...], 1.0)
        out_ref[...] = (jnp.dot(pooled, wfc_ref[...],
                                preferred_element_type=jnp.float32)
                        + bfc_ref[...])


def kernel(x, edge_index, batch, W1, b1, W2, b2, W3, b3, Wfc, bfc):
    N, D = x.shape
    E = edge_index.shape[1]
    H = W1.shape[1]
    C = Wfc.shape[1]
    G = _G

    NP = ((N + 1 + _BR - 1) // _BR) * _BR
    nblocks = NP // _BR
    per_round = _NC * _NS * _CHUNK
    NCH = -(-E // per_round)
    NCH = ((NCH + 7) // 8) * 8
    EP = NCH * per_round
    pad = EP - E
    npad_rows = NP - N

    spread = (jnp.arange(pad, dtype=jnp.int32) % npad_rows) + N
    ei_pad = jnp.concatenate(
        [edge_index, jnp.broadcast_to(spread, (2, pad))], axis=1
    ).reshape(2, _NC, _NS, NCH, _CHUNK)
    s_arr = ei_pad[0]
    d_arr = ei_pad[1]
    batch_2d = batch.reshape(1, N)

    deg_kernel = _make_deg_kernel(NP, NCH)
    agg_kernel = _make_agg_kernel(NP, NCH, H)

    degp = deg_kernel(d_arr)

    hs1, dinv = pl.pallas_call(
        functools.partial(_prep_body, Nreal=N),
        grid=(nblocks,),
        in_specs=[
            pl.BlockSpec((_BR, D), lambda i: (i, 0)),
            pl.BlockSpec((D, H), lambda i: (0, 0)),
            pl.BlockSpec((2, _BR), lambda i: (0, i)),
        ],
        out_specs=[
            pl.BlockSpec((_BR, H), lambda i: (i, 0)),
            pl.BlockSpec((_BR, 1), lambda i: (i, 0)),
        ],
        out_shape=[
            jax.ShapeDtypeStruct((NP, H), jnp.bfloat16),
            jax.ShapeDtypeStruct((NP, 1), jnp.float32),
        ],
    )(x, W1, degp)

    def mid(p, hs, b, W):
        return pl.pallas_call(
            functools.partial(_mid_body, Nreal=N),
            grid=(nblocks,),
            in_specs=[
                pl.BlockSpec((_NC, _BR, H), lambda i: (0, i, 0)),
                pl.BlockSpec((_BR, H), lambda i: (i, 0)),
                pl.BlockSpec((_BR, 1), lambda i: (i, 0)),
                pl.BlockSpec((1, H), lambda i: (0, 0)),
                pl.BlockSpec((H, H), lambda i: (0, 0)),
            ],
            out_specs=pl.BlockSpec((_BR, H), lambda i: (i, 0)),
            out_shape=jax.ShapeDtypeStruct((NP, H), jnp.bfloat16),
        )(p, hs, dinv, b.reshape(1, H), W)

    p1 = agg_kernel(hs1, s_arr, d_arr)
    hs2 = mid(p1, hs1, b1, W2)
    p2 = agg_kernel(hs2, s_arr, d_arr)
    hs3 = mid(p2, hs2, b2, W3)
    p3 = agg_kernel(hs3, s_arr, d_arr)

    out = pl.pallas_call(
        functools.partial(_final_body, nblocks=nblocks, G=G, Nreal=N),
        grid=(nblocks,),
        in_specs=[
            pl.BlockSpec((_NC, _BR, H), lambda i: (0, i, 0)),
            pl.BlockSpec((_BR, H), lambda i: (i, 0)),
            pl.BlockSpec((_BR, 1), lambda i: (i, 0)),
            pl.BlockSpec((1, H), lambda i: (0, 0)),
            pl.BlockSpec((1, _BR), lambda i: (0, i)),
            pl.BlockSpec((H, C), lambda i: (0, 0)),
            pl.BlockSpec((1, C), lambda i: (0, 0)),
        ],
        out_specs=pl.BlockSpec((G, C), lambda i: (0, 0)),
        out_shape=jax.ShapeDtypeStruct((G, C), jnp.float32),
        scratch_shapes=[
            pltpu.VMEM((G, H), jnp.float32),
            pltpu.VMEM((G, 1), jnp.float32),
        ],
    )(p3, hs3, dinv, b3.reshape(1, H), batch_2d, Wfc, bfc.reshape(1, C))

    return out


# speedup vs baseline: 1.4051x; 1.0212x over previous
import functools

import jax
import jax.numpy as jnp
from jax import lax
from jax.experimental import pallas as pl
from jax.experimental.pallas import tpu as pltpu
from jax.experimental.pallas import tpu_sc as plsc

_NC = 2
_NS = 16
_CHUNK = 128
_BR = 2048
_G = 64


def _sc_mesh():
    return plsc.VectorSubcoreMesh(core_axis_name="c", subcore_axis_name="s",
                                  num_cores=_NC, num_subcores=_NS)


def _make_deg_kernel(NP, NCH):
    rpt = NP // _NS

    @functools.partial(
        pl.kernel,
        out_type=jax.ShapeDtypeStruct((_NC, NP), jnp.float32),
        mesh=_sc_mesh(),
        compiler_params=pltpu.CompilerParams(use_tc_tiling_on_sc=False, skip_device_barrier=True, disable_bounds_checks=True, disable_semaphore_checks=True),
        scratch_types=[
            pltpu.VMEM((NCH, _CHUNK), jnp.int32),
            pltpu.VMEM((_CHUNK,), jnp.float32),
            pltpu.VMEM((rpt,), jnp.float32),
            pltpu.VMEM_SHARED((NP,), jnp.float32),
            [pltpu.SemaphoreType.DMA] * 4,
        ],
    )
    def deg_kernel(d_hbm, out_hbm, didx, ones, zbuf, acc, dsem):
        c = lax.axis_index("c")
        t = lax.axis_index("s")
        pltpu.sync_copy(d_hbm.at[c, t], didx)
        one16 = jnp.ones((16,), jnp.float32)
        zero16 = jnp.zeros((16,), jnp.float32)
        for i in range(_CHUNK // 16):
            ones[pl.ds(i * 16, 16)] = one16
        for i in range(rpt // 16):
            zbuf[pl.ds(i * 16, 16)] = zero16
        pltpu.sync_copy(zbuf, acc.at[pl.ds(t * rpt, rpt)])
        plsc.subcore_barrier()

        def body(jj, carry):
            for b in range(4):
                j = 4 * jj + b
                pltpu.async_copy(ones, acc.at[didx.at[j]], dsem[b],
                                 add=True)

                @pl.when(j >= 4)
                def _():
                    pltpu.make_async_copy(ones, acc.at[didx.at[j - 4]],
                                          dsem[b]).wait()
            return carry

        lax.fori_loop(0, NCH // 4, body, 0, unroll=False)
        for k in range(4, 0, -1):
            pltpu.make_async_copy(ones, acc.at[didx.at[NCH - k]],
                                  dsem[(NCH - k) % 4]).wait()
        plsc.subcore_barrier()
        pltpu.sync_copy(acc.at[pl.ds(t * rpt, rpt)],
                        out_hbm.at[c, pl.ds(t * rpt, rpt)])

    return deg_kernel


def _make_agg_kernel(NP, NCH, H):
    rpt = NP // _NS

    @functools.partial(
        pl.kernel,
        out_type=jax.ShapeDtypeStruct((_NC, NP, H), jnp.bfloat16),
        mesh=_sc_mesh(),
        compiler_params=pltpu.CompilerParams(use_tc_tiling_on_sc=False, skip_device_barrier=True, disable_bounds_checks=True, disable_semaphore_checks=True),
        scratch_types=[
            pltpu.VMEM((NCH, _CHUNK), jnp.int32),
            pltpu.VMEM((NCH, _CHUNK), jnp.int32),
            pltpu.VMEM((12, _CHUNK, H), jnp.bfloat16),
            pltpu.VMEM_SHARED((NP, H), jnp.bfloat16),
            [pltpu.SemaphoreType.DMA] * 12,
            [pltpu.SemaphoreType.DMA] * 12,
            pltpu.SemaphoreType.DMA,
        ],
    )
    def agg_kernel(hs_hbm, s_hbm, d_hbm, out_hbm, sidx, didx, rows, acc,
                   gsem, ssem, isem):
        c = lax.axis_index("c")
        t = lax.axis_index("s")

        def gather(j, b):
            return pltpu.make_async_copy(hs_hbm.at[sidx.at[j]], rows.at[b],
                                         gsem[b])

        def scatter_start(j, b):
            pltpu.async_copy(rows.at[b], acc.at[didx.at[j]], ssem[b],
                             add=True)

        def scatter_wait(j, b):
            pltpu.make_async_copy(rows.at[b], acc.at[didx.at[j]],
                                  ssem[b]).wait()

        pltpu.sync_copy(s_hbm.at[c, t], sidx)
        init_cp = pltpu.make_async_copy(hs_hbm.at[pl.ds(t * rpt, rpt)],
                                        acc.at[pl.ds(t * rpt, rpt)], isem)
        init_cp.start()
        for b in range(6):
            gather(b, b).start()
        pltpu.sync_copy(d_hbm.at[c, t], didx)
        init_cp.wait()
        plsc.subcore_barrier()

        def body(jj, carry):
            for b in range(12):
                j = 12 * jj + b
                gather(j, b).wait()
                scatter_start(j, b)
                b6 = (b + 6) % 12
                j6 = j + 6

                @pl.when(j6 >= 12)
                def _():
                    scatter_wait(j6 - 12, b6)

                @pl.when(j6 < NCH)
                def _():
                    gather(j6, b6).start()
            return carry

        lax.fori_loop(0, NCH // 12, body, 0, unroll=False)
        for k in range(6, 0, -1):
            scatter_wait(NCH - k, (NCH - k) % 12)
        plsc.subcore_barrier()
        pltpu.sync_copy(acc.at[pl.ds(t * rpt, rpt)],
                        out_hbm.at[c, pl.ds(t * rpt, rpt)])

    return agg_kernel


def _prep_body(x_ref, w_ref, degp_ref, hs_ref, dinv_ref, *, Nreal):
    i = pl.program_id(0)
    deg = (degp_ref[0:1, :] + degp_ref[1:2, :] + 1.0).reshape(_BR, 1)
    dinv = lax.rsqrt(deg)
    h = jnp.dot(x_ref[...], w_ref[...], preferred_element_type=jnp.float32)
    rows = i * _BR + lax.broadcasted_iota(jnp.int32, (_BR, 1), 0)
    valid = rows < Nreal
    hs_ref[...] = jnp.where(valid, dinv * h, 0.0).astype(jnp.bfloat16)
    dinv_ref[...] = jnp.where(valid, dinv, 1.0)


def _mid_body(p_ref, hs_ref, dinv_ref, b_ref, w_ref, out_ref, *, Nreal):
    i = pl.program_id(0)
    psum = (p_ref[0].astype(jnp.float32) + p_ref[1].astype(jnp.float32)
            - hs_ref[...].astype(jnp.float32))
    z = dinv_ref[...] * psum + b_ref[...]
    h = jnp.maximum(z, 0.0)
    hn = jnp.dot(h, w_ref[...], preferred_element_type=jnp.float32)
    rows = i * _BR + lax.broadcasted_iota(jnp.int32, (_BR, 1), 0)
    out_ref[...] = jnp.where(rows < Nreal, dinv_ref[...] * hn,
                             0.0).astype(jnp.bfloat16)


def _final_body(p_ref, hs_ref, dinv_ref, b_ref, batch_ref, wfc_ref, bfc_ref,
                out_ref, pooled_acc, cnt_acc, *, nblocks, G, Nreal):
    i = pl.program_id(0)

    @pl.when(i == 0)
    def _():
        pooled_acc[...] = jnp.zeros_like(pooled_acc)
        cnt_acc[...] = jnp.zeros_like(cnt_acc)

    psum = (p_ref[0].astype(jnp.float32) + p_ref[1].astype(jnp.float32)
            - hs_ref[...].astype(jnp.float32))
    z = dinv_ref[...] * psum + b_ref[...]
    gids = lax.broadcasted_iota(jnp.int32, (G, _BR), 0)
    cols = i * _BR + lax.broadcasted_iota(jnp.int32, (G, _BR), 1)
    mask = ((gids == batch_ref[...]) & (cols < Nreal)
            ).astype(jnp.float32)
    pooled_acc[...] += jnp.dot(mask, z, preferred_element_type=jnp.float32)
    cnt_acc[...] += jnp.sum(mask, axis=1, keepdims=True)

    @pl.when(i == nblocks - 1)
    def _():
        pooled = pooled_acc[...] / jnp.maximum(cnt_acc[...], 1.0)
        out_ref[...] = (jnp.dot(pooled, wfc_ref[...],
                                preferred_element_type=jnp.float32)
                        + bfc_ref[...])


def kernel(x, edge_index, batch, W1, b1, W2, b2, W3, b3, Wfc, bfc):
    N, D = x.shape
    E = edge_index.shape[1]
    H = W1.shape[1]
    C = Wfc.shape[1]
    G = _G

    NP = ((N + 1 + _BR - 1) // _BR) * _BR
    nblocks = NP // _BR
    per_round = _NC * _NS * _CHUNK
    NCH = -(-E // per_round)
    NCH = ((NCH + 11) // 12) * 12
    EP = NCH * per_round
    pad = EP - E
    npad_rows = NP - N

    spread = (jnp.arange(pad, dtype=jnp.int32) % npad_rows) + N
    ei_pad = jnp.concatenate(
        [edge_index, jnp.broadcast_to(spread, (2, pad))], axis=1
    ).reshape(2, _NC, _NS, NCH, _CHUNK)
    s_arr = ei_pad[0]
    d_arr = ei_pad[1]
    batch_2d = batch.reshape(1, N)

    deg_kernel = _make_deg_kernel(NP, NCH)
    agg_kernel = _make_agg_kernel(NP, NCH, H)

    degp = deg_kernel(d_arr)

    hs1, dinv = pl.pallas_call(
        functools.partial(_prep_body, Nreal=N),
        grid=(nblocks,),
        in_specs=[
            pl.BlockSpec((_BR, D), lambda i: (i, 0)),
            pl.BlockSpec((D, H), lambda i: (0, 0)),
            pl.BlockSpec((2, _BR), lambda i: (0, i)),
        ],
        out_specs=[
            pl.BlockSpec((_BR, H), lambda i: (i, 0)),
            pl.BlockSpec((_BR, 1), lambda i: (i, 0)),
        ],
        out_shape=[
            jax.ShapeDtypeStruct((NP, H), jnp.bfloat16),
            jax.ShapeDtypeStruct((NP, 1), jnp.float32),
        ],
    )(x, W1, degp)

    def mid(p, hs, b, W):
        return pl.pallas_call(
            functools.partial(_mid_body, Nreal=N),
            grid=(nblocks,),
            in_specs=[
                pl.BlockSpec((_NC, _BR, H), lambda i: (0, i, 0)),
                pl.BlockSpec((_BR, H), lambda i: (i, 0)),
                pl.BlockSpec((_BR, 1), lambda i: (i, 0)),
                pl.BlockSpec((1, H), lambda i: (0, 0)),
                pl.BlockSpec((H, H), lambda i: (0, 0)),
            ],
            out_specs=pl.BlockSpec((_BR, H), lambda i: (i, 0)),
            out_shape=jax.ShapeDtypeStruct((NP, H), jnp.bfloat16),
        )(p, hs, dinv, b.reshape(1, H), W)

    p1 = agg_kernel(hs1, s_arr, d_arr)
    hs2 = mid(p1, hs1, b1, W2)
    p2 = agg_kernel(hs2, s_arr, d_arr)
    hs3 = mid(p2, hs2, b2, W3)
    p3 = agg_kernel(hs3, s_arr, d_arr)

    out = pl.pallas_call(
        functools.partial(_final_body, nblocks=nblocks, G=G, Nreal=N),
        grid=(nblocks,),
        in_specs=[
            pl.BlockSpec((_NC, _BR, H), lambda i: (0, i, 0)),
            pl.BlockSpec((_BR, H), lambda i: (i, 0)),
            pl.BlockSpec((_BR, 1), lambda i: (i, 0)),
            pl.BlockSpec((1, H), lambda i: (0, 0)),
            pl.BlockSpec((1, _BR), lambda i: (0, i)),
            pl.BlockSpec((H, C), lambda i: (0, 0)),
            pl.BlockSpec((1, C), lambda i: (0, 0)),
        ],
        out_specs=pl.BlockSpec((G, C), lambda i: (0, 0)),
        out_shape=jax.ShapeDtypeStruct((G, C), jnp.float32),
        scratch_shapes=[
            pltpu.VMEM((G, H), jnp.float32),
            pltpu.VMEM((G, 1), jnp.float32),
        ],
    )(p3, hs3, dinv, b3.reshape(1, H), batch_2d, Wfc, bfc.reshape(1, C))

    return out
